# baseline (speedup 1.0000x reference)
import functools

import jax
import jax.numpy as jnp
from jax import lax
from jax.experimental import pallas as pl
from jax.experimental.pallas import tpu as pltpu
from jax.experimental.pallas import tpu_sc as plsc

N_NODES = 10000
D_FEAT = 128
N_EDGES = 320000

NC = 2
NS = 16
NW = NC * NS
CHUNK = 128
NCHUNK = 80
E_PAD = NW * NCHUNK * CHUNK
NROWS = E_PAD // CHUNK
RPT = N_NODES // NS
NCH0 = 96
NCH1 = 64
QCH = 32


def _sc_sums(feature, src_flat, dst_flat):
    mesh = plsc.VectorSubcoreMesh(core_axis_name="c", subcore_axis_name="s")

    @functools.partial(
        pl.kernel,
        out_type=jax.ShapeDtypeStruct((NC, NS, RPT, D_FEAT), jnp.float32),
        mesh=mesh,
        scratch_types=(
            pltpu.VMEM_SHARED((N_NODES + 8, D_FEAT), jnp.float32),
            pltpu.VMEM((CHUNK,), jnp.int32),
            pltpu.VMEM((CHUNK,), jnp.int32),
            pltpu.VMEM((CHUNK, D_FEAT), jnp.float32),
            pltpu.SemaphoreType.DMA,
        ),
    )
    def sc_fn(feature_hbm, src_hbm, dst_hbm, sum_out,
              acc_sh, src_c, dst_c, rows_v, sem):
        c = lax.axis_index("c")
        s = lax.axis_index("s")

        zvec = jnp.zeros((16,), jnp.float32)

        def zrbody(i, _):
            rows_v[i // 8, pl.ds((i % 8) * 16, 16)] = zvec
            return 0
        lax.fori_loop(0, CHUNK * 8, zrbody, 0)

        for j in range(4):
            pltpu.sync_copy(rows_v, acc_sh.at[pl.ds(s * RPT + j * CHUNK, CHUNK)])
        pltpu.sync_copy(rows_v.at[pl.ds(0, RPT - 4 * CHUNK)],
                        acc_sh.at[pl.ds(s * RPT + 4 * CHUNK, RPT - 4 * CHUNK)])
        plsc.subcore_barrier()

        ebase = jnp.where(c == 0, s * NCH0, NS * NCH0 + s * NCH1) * CHUNK
        n_chunks = jnp.where(c == 0, NCH0, NCH1)

        def body(t, _):
            pltpu.sync_copy(src_hbm.at[pl.ds(ebase + t * CHUNK, CHUNK)], src_c)
            pltpu.sync_copy(dst_hbm.at[pl.ds(ebase + t * CHUNK, CHUNK)], dst_c)
            pltpu.async_copy(feature_hbm.at[src_c], rows_v, sem).wait()
            pltpu.sync_copy(rows_v, acc_sh.at[dst_c], add=True)
            return 0

        lax.fori_loop(0, n_chunks, body, 0)
        plsc.subcore_barrier()
        pltpu.sync_copy(acc_sh.at[pl.ds(s * RPT, RPT)], sum_out.at[c, s])

    return sc_fn(feature, src_flat, dst_flat)


def _sc_counts(dst3):
    mesh = plsc.VectorSubcoreMesh(core_axis_name="c", subcore_axis_name="s")

    @functools.partial(
        pl.kernel,
        out_type=jax.ShapeDtypeStruct((NC, NS, RPT, D_FEAT), jnp.float32),
        mesh=mesh,
        scratch_types=(
            pltpu.VMEM_SHARED((N_NODES + 8, D_FEAT), jnp.float32),
            pltpu.VMEM((CHUNK, D_FEAT), jnp.float32),
            pltpu.VMEM((NCHUNK, CHUNK), jnp.int32),
        ),
    )
    def sc_fn(dst_hbm, cnt_out, cnt_sh, ones_v, dst_all):
        c = lax.axis_index("c")
        s = lax.axis_index("s")
        wid = c * NS + s

        zvec = jnp.zeros((16,), jnp.float32)
        ovec = jnp.ones((16,), jnp.float32)

        def zbody(i, _):
            ones_v[i // 8, pl.ds((i % 8) * 16, 16)] = zvec
            return 0
        lax.fori_loop(0, CHUNK * 8, zbody, 0)

        for j in range(4):
            pltpu.sync_copy(ones_v,
                            cnt_sh.at[pl.ds(s * RPT + j * CHUNK, CHUNK)])
        pltpu.sync_copy(ones_v.at[pl.ds(0, RPT - 4 * CHUNK)],
                        cnt_sh.at[pl.ds(s * RPT + 4 * CHUNK, RPT - 4 * CHUNK)])

        def obody(i, _):
            ones_v[i // 8, pl.ds((i % 8) * 16, 16)] = ovec
            return 0
        lax.fori_loop(0, CHUNK * 8, obody, 0)

        pltpu.sync_copy(dst_hbm.at[wid], dst_all)
        plsc.subcore_barrier()

        def cbody(t, _):
            pltpu.sync_copy(ones_v, cnt_sh.at[dst_all.at[t]], add=True)
            return 0

        lax.fori_loop(0, NCHUNK, cbody, 0)
        plsc.subcore_barrier()
        pltpu.sync_copy(cnt_sh.at[pl.ds(s * RPT, RPT)], cnt_out.at[c, s])

    return sc_fn(dst3)


def _tc_body(s0_ref, s1_ref, c0_ref, c1_ref, f_ref, wh_ref, wf_ref, b_ref, o_ref):
    cnt = c0_ref[...][:, 0] + c1_ref[...][:, 0]
    h = (s0_ref[...] + s1_ref[...]) / jnp.maximum(cnt, 1.0)[:, None]
    y = (jnp.dot(h, wh_ref[...], preferred_element_type=jnp.float32)
         + jnp.dot(f_ref[...], wf_ref[...], preferred_element_type=jnp.float32)
         + b_ref[...])
    o_ref[...] = jnp.maximum(y, 0.0)


def _tc_dense(s0, s1, c0, c1, feature, wh, wf, b):
    blk = 1000
    grid = N_NODES // blk
    return pl.pallas_call(
        _tc_body,
        out_shape=jax.ShapeDtypeStruct((N_NODES, D_FEAT), jnp.float32),
        grid=(grid,),
        in_specs=[
            pl.BlockSpec((blk, D_FEAT), lambda i: (i, 0)),
            pl.BlockSpec((blk, D_FEAT), lambda i: (i, 0)),
            pl.BlockSpec((blk, D_FEAT), lambda i: (i, 0)),
            pl.BlockSpec((blk, D_FEAT), lambda i: (i, 0)),
            pl.BlockSpec((blk, D_FEAT), lambda i: (i, 0)),
            pl.BlockSpec((D_FEAT, D_FEAT), lambda i: (0, 0)),
            pl.BlockSpec((D_FEAT, D_FEAT), lambda i: (0, 0)),
            pl.BlockSpec((1, D_FEAT), lambda i: (0, 0)),
        ],
        out_specs=pl.BlockSpec((blk, D_FEAT), lambda i: (i, 0)),
    )(s0, s1, c0, c1, feature, wh, wf, b)


def kernel(feature, edge_index, W, b):
    ei = edge_index.astype(jnp.int32)
    npad = E_PAD - N_EDGES
    src_flat = jnp.concatenate([ei[0], jnp.zeros((npad,), jnp.int32)])
    dst_flat = jnp.concatenate([ei[1], jnp.full((npad,), N_NODES, jnp.int32)])
    dst3 = dst_flat.reshape(NW, NCHUNK, CHUNK)

    sums = _sc_sums(feature, src_flat, dst_flat)
    cnts = _sc_counts(dst3)
    sums = sums.reshape(NC, N_NODES, D_FEAT)
    cnts = cnts.reshape(NC, N_NODES, D_FEAT)

    wh = W[:, :D_FEAT].T
    wf = W[:, D_FEAT:].T
    return _tc_dense(sums[0], sums[1], cnts[0], cnts[1], feature, wh, wf,
                     b.reshape(1, D_FEAT))


# speedup vs baseline: 1.3272x; 1.3272x over previous
import functools

import jax
import jax.numpy as jnp
from jax import lax
from jax.experimental import pallas as pl
from jax.experimental.pallas import tpu as pltpu
from jax.experimental.pallas import tpu_sc as plsc

N_NODES = 10000
D_FEAT = 128
N_EDGES = 320000

NC = 2
NS = 16
NW = NC * NS
CHUNK = 128
NCHUNK = 79
EPW = NCHUNK * CHUNK
E_PAD = NW * EPW
RPT = N_NODES // NS


def _sc_sums(feature, src_flat, dst_flat):
    mesh = plsc.VectorSubcoreMesh(core_axis_name="c", subcore_axis_name="s")

    @functools.partial(
        pl.kernel,
        out_type=jax.ShapeDtypeStruct((NC, NS, RPT, D_FEAT), jnp.float32),
        mesh=mesh,
        scratch_types=(
            pltpu.VMEM_SHARED((N_NODES + 8, D_FEAT), jnp.float32),
            pltpu.VMEM((CHUNK,), jnp.int32),
            pltpu.VMEM((CHUNK,), jnp.int32),
            pltpu.VMEM((CHUNK, D_FEAT), jnp.float32),
            pltpu.SemaphoreType.DMA,
        ),
    )
    def sc_fn(feature_hbm, src_hbm, dst_hbm, sum_out,
              acc_sh, src_c, dst_c, rows_v, sem):
        c = lax.axis_index("c")
        s = lax.axis_index("s")

        zvec = jnp.zeros((16,), jnp.float32)

        def zrbody(i, _):
            rows_v[i // 8, pl.ds((i % 8) * 16, 16)] = zvec
            return 0
        lax.fori_loop(0, CHUNK * 8, zrbody, 0)

        for j in range(4):
            pltpu.sync_copy(rows_v, acc_sh.at[pl.ds(s * RPT + j * CHUNK, CHUNK)])
        pltpu.sync_copy(rows_v.at[pl.ds(0, RPT - 4 * CHUNK)],
                        acc_sh.at[pl.ds(s * RPT + 4 * CHUNK, RPT - 4 * CHUNK)])
        plsc.subcore_barrier()

        ebase = (c * NS + s) * EPW

        def body(t, _):
            pltpu.sync_copy(src_hbm.at[pl.ds(ebase + t * CHUNK, CHUNK)], src_c)
            pltpu.sync_copy(dst_hbm.at[pl.ds(ebase + t * CHUNK, CHUNK)], dst_c)
            pltpu.async_copy(feature_hbm.at[src_c], rows_v, sem).wait()
            pltpu.sync_copy(rows_v, acc_sh.at[dst_c], add=True)
            return 0

        lax.fori_loop(0, NCHUNK, body, 0)
        plsc.subcore_barrier()
        pltpu.sync_copy(acc_sh.at[pl.ds(s * RPT, RPT)], sum_out.at[c, s])

    return sc_fn(feature, src_flat, dst_flat)


def _sc_counts(dst3):
    mesh = plsc.VectorSubcoreMesh(core_axis_name="c", subcore_axis_name="s")

    @functools.partial(
        pl.kernel,
        out_type=jax.ShapeDtypeStruct((NC, NS, RPT, D_FEAT), jnp.float32),
        mesh=mesh,
        scratch_types=(
            pltpu.VMEM_SHARED((N_NODES + 8, D_FEAT), jnp.float32),
            pltpu.VMEM((CHUNK, D_FEAT), jnp.float32),
            pltpu.VMEM((NCHUNK, CHUNK), jnp.int32),
        ),
    )
    def sc_fn(dst_hbm, cnt_out, cnt_sh, ones_v, dst_all):
        c = lax.axis_index("c")
        s = lax.axis_index("s")
        wid = c * NS + s

        zvec = jnp.zeros((16,), jnp.float32)
        ovec = jnp.ones((16,), jnp.float32)

        def zbody(i, _):
            ones_v[i // 8, pl.ds((i % 8) * 16, 16)] = zvec
            return 0
        lax.fori_loop(0, CHUNK * 8, zbody, 0)

        for j in range(4):
            pltpu.sync_copy(ones_v,
                            cnt_sh.at[pl.ds(s * RPT + j * CHUNK, CHUNK)])
        pltpu.sync_copy(ones_v.at[pl.ds(0, RPT - 4 * CHUNK)],
                        cnt_sh.at[pl.ds(s * RPT + 4 * CHUNK, RPT - 4 * CHUNK)])

        def obody(i, _):
            ones_v[i // 8, pl.ds((i % 8) * 16, 16)] = ovec
            return 0
        lax.fori_loop(0, CHUNK * 8, obody, 0)

        pltpu.sync_copy(dst_hbm.at[wid], dst_all)
        plsc.subcore_barrier()

        def cbody(t, _):
            pltpu.sync_copy(ones_v, cnt_sh.at[dst_all.at[t]], add=True)
            return 0

        lax.fori_loop(0, NCHUNK, cbody, 0)
        plsc.subcore_barrier()
        pltpu.sync_copy(cnt_sh.at[pl.ds(s * RPT, RPT)], cnt_out.at[c, s])

    return sc_fn(dst3)


def _tc_body(s0_ref, s1_ref, c0_ref, c1_ref, f_ref, wh_ref, wf_ref, b_ref, o_ref):
    cnt = c0_ref[...][:, 0] + c1_ref[...][:, 0]
    h = (s0_ref[...] + s1_ref[...]) / jnp.maximum(cnt, 1.0)[:, None]
    y = (jnp.dot(h, wh_ref[...], preferred_element_type=jnp.float32)
         + jnp.dot(f_ref[...], wf_ref[...], preferred_element_type=jnp.float32)
         + b_ref[...])
    o_ref[...] = jnp.maximum(y, 0.0)


def _tc_dense(s0, s1, c0, c1, feature, wh, wf, b):
    blk = 1000
    grid = N_NODES // blk
    return pl.pallas_call(
        _tc_body,
        out_shape=jax.ShapeDtypeStruct((N_NODES, D_FEAT), jnp.float32),
        grid=(grid,),
        in_specs=[
            pl.BlockSpec((blk, D_FEAT), lambda i: (i, 0)),
            pl.BlockSpec((blk, D_FEAT), lambda i: (i, 0)),
            pl.BlockSpec((blk, D_FEAT), lambda i: (i, 0)),
            pl.BlockSpec((blk, D_FEAT), lambda i: (i, 0)),
            pl.BlockSpec((blk, D_FEAT), lambda i: (i, 0)),
            pl.BlockSpec((D_FEAT, D_FEAT), lambda i: (0, 0)),
            pl.BlockSpec((D_FEAT, D_FEAT), lambda i: (0, 0)),
            pl.BlockSpec((1, D_FEAT), lambda i: (0, 0)),
        ],
        out_specs=pl.BlockSpec((blk, D_FEAT), lambda i: (i, 0)),
    )(s0, s1, c0, c1, feature, wh, wf, b)


def kernel(feature, edge_index, W, b):
    ei = edge_index.astype(jnp.int32)
    npad = E_PAD - N_EDGES
    src_flat = jnp.concatenate([ei[0], jnp.zeros((npad,), jnp.int32)])
    dst_flat = jnp.concatenate([ei[1], jnp.full((npad,), N_NODES, jnp.int32)])
    dst3 = dst_flat.reshape(NW, NCHUNK, CHUNK)

    sums = _sc_sums(feature, src_flat, dst_flat)
    cnts = _sc_counts(dst3)
    sums = sums.reshape(NC, N_NODES, D_FEAT)
    cnts = cnts.reshape(NC, N_NODES, D_FEAT)

    wh = W[:, :D_FEAT].T
    wf = W[:, D_FEAT:].T
    return _tc_dense(sums[0], sums[1], cnts[0], cnts[1], feature, wh, wf,
                     b.reshape(1, D_FEAT))


# speedup vs baseline: 1.4096x; 1.0621x over previous
import functools

import jax
import jax.numpy as jnp
from jax import lax
from jax.experimental import pallas as pl
from jax.experimental.pallas import tpu as pltpu
from jax.experimental.pallas import tpu_sc as plsc

N_NODES = 10000
D_FEAT = 128
N_EDGES = 320000

NC = 2
NS = 16
NW = NC * NS
CHUNK = 128
NCHUNK = 79
EPW = NCHUNK * CHUNK
E_PAD = NW * EPW
RPT = N_NODES // NS


def _sc_sums(feature, src_flat, dst_flat):
    mesh = plsc.VectorSubcoreMesh(core_axis_name="c", subcore_axis_name="s")

    @functools.partial(
        pl.kernel,
        out_type=jax.ShapeDtypeStruct((NC, NS, RPT, D_FEAT), jnp.float32),
        mesh=mesh,
        scratch_types=(
            pltpu.VMEM_SHARED((N_NODES + 8, D_FEAT), jnp.float32),
            pltpu.VMEM((CHUNK,), jnp.int32),
            pltpu.VMEM((CHUNK,), jnp.int32),
            pltpu.VMEM((CHUNK, D_FEAT), jnp.float32),
            pltpu.SemaphoreType.DMA,
        ),
    )
    def sc_fn(feature_hbm, src_hbm, dst_hbm, sum_out,
              acc_sh, src_c, dst_c, rows_v, sem):
        c = lax.axis_index("c")
        s = lax.axis_index("s")

        zvec = jnp.zeros((16,), jnp.float32)

        def zrbody(i, _):
            rows_v[i // 8, pl.ds((i % 8) * 16, 16)] = zvec
            return 0
        lax.fori_loop(0, CHUNK * 8, zrbody, 0)

        for j in range(4):
            pltpu.sync_copy(rows_v, acc_sh.at[pl.ds(s * RPT + j * CHUNK, CHUNK)])
        pltpu.sync_copy(rows_v.at[pl.ds(0, RPT - 4 * CHUNK)],
                        acc_sh.at[pl.ds(s * RPT + 4 * CHUNK, RPT - 4 * CHUNK)])
        plsc.subcore_barrier()

        ebase = (c * NS + s) * EPW

        def body(t, _):
            pltpu.sync_copy(src_hbm.at[pl.ds(ebase + t * CHUNK, CHUNK)], src_c)
            cp = pltpu.async_copy(feature_hbm.at[src_c], rows_v, sem)
            pltpu.sync_copy(dst_hbm.at[pl.ds(ebase + t * CHUNK, CHUNK)], dst_c)
            cp.wait()
            pltpu.sync_copy(rows_v, acc_sh.at[dst_c], add=True)
            return 0

        lax.fori_loop(0, NCHUNK, body, 0)
        plsc.subcore_barrier()
        pltpu.sync_copy(acc_sh.at[pl.ds(s * RPT, RPT)], sum_out.at[c, s])

    return sc_fn(feature, src_flat, dst_flat)


def _sc_counts(dst3):
    mesh = plsc.VectorSubcoreMesh(core_axis_name="c", subcore_axis_name="s")

    @functools.partial(
        pl.kernel,
        out_type=jax.ShapeDtypeStruct((NC, NS, RPT, D_FEAT), jnp.float32),
        mesh=mesh,
        scratch_types=(
            pltpu.VMEM_SHARED((N_NODES + 8, D_FEAT), jnp.float32),
            pltpu.VMEM((CHUNK, D_FEAT), jnp.float32),
            pltpu.VMEM((NCHUNK, CHUNK), jnp.int32),
        ),
    )
    def sc_fn(dst_hbm, cnt_out, cnt_sh, ones_v, dst_all):
        c = lax.axis_index("c")
        s = lax.axis_index("s")
        wid = c * NS + s

        zvec = jnp.zeros((16,), jnp.float32)
        ovec = jnp.ones((16,), jnp.float32)

        def zbody(i, _):
            ones_v[i // 8, pl.ds((i % 8) * 16, 16)] = zvec
            return 0
        lax.fori_loop(0, CHUNK * 8, zbody, 0)

        for j in range(4):
            pltpu.sync_copy(ones_v,
                            cnt_sh.at[pl.ds(s * RPT + j * CHUNK, CHUNK)])
        pltpu.sync_copy(ones_v.at[pl.ds(0, RPT - 4 * CHUNK)],
                        cnt_sh.at[pl.ds(s * RPT + 4 * CHUNK, RPT - 4 * CHUNK)])

        def obody(i, _):
            ones_v[i // 8, pl.ds((i % 8) * 16, 16)] = ovec
            return 0
        lax.fori_loop(0, CHUNK * 8, obody, 0)

        pltpu.sync_copy(dst_hbm.at[wid], dst_all)
        plsc.subcore_barrier()

        def cbody(t, _):
            pltpu.sync_copy(ones_v, cnt_sh.at[dst_all.at[t]], add=True)
            return 0

        lax.fori_loop(0, NCHUNK, cbody, 0)
        plsc.subcore_barrier()
        pltpu.sync_copy(cnt_sh.at[pl.ds(s * RPT, RPT)], cnt_out.at[c, s])

    return sc_fn(dst3)


def _tc_body(s0_ref, s1_ref, c0_ref, c1_ref, f_ref, wh_ref, wf_ref, b_ref, o_ref):
    cnt = c0_ref[...][:, 0] + c1_ref[...][:, 0]
    h = (s0_ref[...] + s1_ref[...]) / jnp.maximum(cnt, 1.0)[:, None]
    y = (jnp.dot(h, wh_ref[...], preferred_element_type=jnp.float32)
         + jnp.dot(f_ref[...], wf_ref[...], preferred_element_type=jnp.float32)
         + b_ref[...])
    o_ref[...] = jnp.maximum(y, 0.0)


def _tc_dense(s0, s1, c0, c1, feature, wh, wf, b):
    blk = 1000
    grid = N_NODES // blk
    return pl.pallas_call(
        _tc_body,
        out_shape=jax.ShapeDtypeStruct((N_NODES, D_FEAT), jnp.float32),
        grid=(grid,),
        in_specs=[
            pl.BlockSpec((blk, D_FEAT), lambda i: (i, 0)),
            pl.BlockSpec((blk, D_FEAT), lambda i: (i, 0)),
            pl.BlockSpec((blk, D_FEAT), lambda i: (i, 0)),
            pl.BlockSpec((blk, D_FEAT), lambda i: (i, 0)),
            pl.BlockSpec((blk, D_FEAT), lambda i: (i, 0)),
            pl.BlockSpec((D_FEAT, D_FEAT), lambda i: (0, 0)),
            pl.BlockSpec((D_FEAT, D_FEAT), lambda i: (0, 0)),
            pl.BlockSpec((1, D_FEAT), lambda i: (0, 0)),
        ],
        out_specs=pl.BlockSpec((blk, D_FEAT), lambda i: (i, 0)),
    )(s0, s1, c0, c1, feature, wh, wf, b)


def kernel(feature, edge_index, W, b):
    ei = edge_index.astype(jnp.int32)
    npad = E_PAD - N_EDGES
    src_flat = jnp.concatenate([ei[0], jnp.zeros((npad,), jnp.int32)])
    dst_flat = jnp.concatenate([ei[1], jnp.full((npad,), N_NODES, jnp.int32)])
    dst3 = dst_flat.reshape(NW, NCHUNK, CHUNK)

    sums = _sc_sums(feature, src_flat, dst_flat)
    cnts = _sc_counts(dst3)
    sums = sums.reshape(NC, N_NODES, D_FEAT)
    cnts = cnts.reshape(NC, N_NODES, D_FEAT)

    wh = W[:, :D_FEAT].T
    wf = W[:, D_FEAT:].T
    return _tc_dense(sums[0], sums[1], cnts[0], cnts[1], feature, wh, wf,
                     b.reshape(1, D_FEAT))
